# Initial kernel scaffold; baseline (speedup 1.0000x reference)
import jax
import jax.numpy as jnp
from jax.experimental import pallas as pl


def kernel(x, points, values):
    raise NotImplementedError("write your pallas kernel here")


# speedup vs baseline: 19.9928x; 19.9928x over previous
import functools

import jax
import jax.numpy as jnp
from jax import lax
from jax.experimental import pallas as pl
from jax.experimental.pallas import tpu as pltpu
from jax.experimental.pallas import tpu_sc as plsc

N_KNOTS = 4096
N_QUERIES = 16384
NC = 2
NS = 16
LANES = 16
NW = NC * NS
QPW = N_QUERIES // NW
GROUPS = QPW // LANES
MM_CHUNK = N_QUERIES // NS
MM_STEPS = MM_CHUNK // LANES
SEARCH_STEPS = 12


def _lower_bound(pts_v, target, mid0_splat):
    mid0 = N_KNOTS // 2
    less = mid0_splat < target
    lo = jnp.where(less, mid0 + 1, 0)
    hi = jnp.where(less, N_KNOTS, mid0)
    for _ in range(SEARCH_STEPS - 1):
        mid = lax.shift_right_logical(lo + hi, 1)
        pv = plsc.load_gather(pts_v, [mid])
        less = pv < target
        lo = jnp.where(less, mid + 1, lo)
        hi = jnp.where(less, hi, mid)
    return lo


def _recip(d):
    r = 1.0 / d
    r = r * (2.0 - d * r)
    r = r * (2.0 - d * r)
    return r


def _vec_interp(xs, x0, x1, y0, y1):
    delta = x1 - x0
    deg = delta == 0.0
    delta = jnp.where(deg, 1.0, delta)
    return jnp.where(deg, y0, y0 + (y1 - y0) * (xs - x0) * _recip(delta))


def _body(x_hbm, pts_hbm, vals_hbm, out_hbm,
          x_v, pts_v, vals_v, out_v):
    cid = lax.axis_index("c")
    sid = lax.axis_index("s")
    wid = sid * NC + cid

    pltpu.sync_copy(x_hbm, x_v)
    pltpu.sync_copy(pts_hbm, pts_v)
    pltpu.sync_copy(vals_hbm, vals_v)

    def mm_step(i, carry):
        vmin, vmax = carry
        xv = x_v[pl.ds(i * LANES, LANES)]
        return jnp.minimum(vmin, xv), jnp.maximum(vmax, xv)

    vmin0 = jnp.full((LANES,), jnp.inf, jnp.float32)
    vmax0 = jnp.full((LANES,), -jnp.inf, jnp.float32)
    vmin, vmax = lax.fori_loop(0, N_QUERIES // LANES, mm_step, (vmin0, vmax0))
    min_x = jnp.broadcast_to(jnp.min(vmin), (LANES,))
    max_x = jnp.broadcast_to(jnp.max(vmax), (LANES,))

    p_head = pts_v[pl.ds(0, LANES)]
    p_tail = pts_v[pl.ds(N_KNOTS - LANES, LANES)]
    v_head = vals_v[pl.ds(0, LANES)]
    v_tail = vals_v[pl.ds(N_KNOTS - LANES, LANES)]

    def splat(vec, lane):
        return jnp.broadcast_to(vec[lane], (LANES,))

    p0 = splat(p_head, 0)
    p1 = splat(p_head, 1)
    pm2 = splat(p_tail, LANES - 2)
    pm1 = splat(p_tail, LANES - 1)
    v0 = splat(v_head, 0)
    v1 = splat(v_head, 1)
    vm2 = splat(v_tail, LANES - 2)
    vm1 = splat(v_tail, LANES - 1)
    lo_val = _vec_interp(min_x, p0, p1, v0, v1)
    hi_val = _vec_interp(max_x, pm2, pm1, vm2, vm1)
    mid0_splat = splat(pts_v[pl.ds(N_KNOTS // 2, LANES)], 0)

    qbase = wid * QPW

    def group(g, carry):
        xq = x_v[pl.ds(qbase + g * LANES, LANES)]
        s = _lower_bound(pts_v, xq, mid0_splat)
        s_c = jnp.minimum(s, N_KNOTS - 1)
        ps = plsc.load_gather(pts_v, [s_c])
        ys = plsc.load_gather(vals_v, [s_c])
        sm1 = jnp.maximum(s - 1, 0)
        pv0 = plsc.load_gather(pts_v, [sm1])
        f = _lower_bound(pts_v, pv0, mid0_splat)
        yf = plsc.load_gather(vals_v, [f])
        in_range = s < N_KNOTS
        equal = jnp.logical_and(in_range, ps == xq)
        case_b = jnp.logical_and(s == 0, jnp.logical_not(equal))
        case_c = jnp.logical_not(in_range)
        x0 = jnp.where(case_b, min_x, pv0)
        y0 = jnp.where(case_b, lo_val, yf)
        x1 = jnp.where(case_b, p0, jnp.where(case_c, max_x, ps))
        y1 = jnp.where(case_b, v0, jnp.where(case_c, hi_val, ys))
        y = jnp.where(equal, ys, y0 + (y1 - y0) * (xq - x0) * _recip(x1 - x0))
        out_v[pl.ds(g * LANES, LANES)] = y
        return carry

    lax.fori_loop(0, GROUPS, group, 0)
    pltpu.sync_copy(out_v, out_hbm.at[pl.ds(qbase, QPW)])


_interp_call = functools.partial(
    pl.kernel,
    mesh=plsc.VectorSubcoreMesh(core_axis_name="c", subcore_axis_name="s"),
    out_type=jax.ShapeDtypeStruct((N_QUERIES,), jnp.float32),
    compiler_params=pltpu.CompilerParams(needs_layout_passes=False),
    scratch_types=[
        pltpu.VMEM((N_QUERIES,), jnp.float32),
        pltpu.VMEM((N_KNOTS,), jnp.float32),
        pltpu.VMEM((N_KNOTS,), jnp.float32),
        pltpu.VMEM((QPW,), jnp.float32),
    ],
)(_body)


def kernel(x, points, values):
    pts, vals = lax.sort([points, values], num_keys=1, is_stable=True)
    return _interp_call(x, pts, vals)


# speedup vs baseline: 26.0904x; 1.3050x over previous
import functools

import jax
import jax.numpy as jnp
from jax import lax
from jax.experimental import pallas as pl
from jax.experimental.pallas import tpu as pltpu
from jax.experimental.pallas import tpu_sc as plsc

N_KNOTS = 4096
N_QUERIES = 16384
NC = 2
NS = 16
LANES = 16
NW = NC * NS
QPW = N_QUERIES // NW
GROUPS = QPW // LANES
MM_CHUNK = N_QUERIES // NS
MM_STEPS = MM_CHUNK // LANES
SEARCH_STEPS = 12


def _lower_bound_multi(pts_v, targets, mid0_splat):
    mid0 = N_KNOTS // 2
    los, his = [], []
    for t in targets:
        less = mid0_splat < t
        los.append(jnp.where(less, mid0 + 1, 0))
        his.append(jnp.where(less, N_KNOTS, mid0))
    for _ in range(SEARCH_STEPS - 1):
        for k in range(len(targets)):
            mid = lax.shift_right_logical(los[k] + his[k], 1)
            pv = plsc.load_gather(pts_v, [mid])
            less = pv < targets[k]
            los[k] = jnp.where(less, mid + 1, los[k])
            his[k] = jnp.where(less, his[k], mid)
    return los


def _recip(d):
    r = 1.0 / d
    r = r * (2.0 - d * r)
    r = r * (2.0 - d * r)
    return r


def _vec_interp(xs, x0, x1, y0, y1):
    delta = x1 - x0
    deg = delta == 0.0
    delta = jnp.where(deg, 1.0, delta)
    return jnp.where(deg, y0, y0 + (y1 - y0) * (xs - x0) * _recip(delta))


def _body(x_hbm, pts_hbm, vals_hbm, out_hbm,
          x_v, pts_v, vals_v, out_v):
    cid = lax.axis_index("c")
    sid = lax.axis_index("s")
    wid = sid * NC + cid

    pltpu.sync_copy(x_hbm, x_v)
    pltpu.sync_copy(pts_hbm, pts_v)
    pltpu.sync_copy(vals_hbm, vals_v)

    MM_UNROLL = 8

    def mm_step(i, carry):
        vmins, vmaxs = carry
        new_mins, new_maxs = [], []
        for k in range(MM_UNROLL):
            xv = x_v[pl.ds((i * MM_UNROLL + k) * LANES, LANES)]
            new_mins.append(jnp.minimum(vmins[k], xv))
            new_maxs.append(jnp.maximum(vmaxs[k], xv))
        return tuple(new_mins), tuple(new_maxs)

    vmin0 = tuple(jnp.full((LANES,), jnp.inf, jnp.float32)
                  for _ in range(MM_UNROLL))
    vmax0 = tuple(jnp.full((LANES,), -jnp.inf, jnp.float32)
                  for _ in range(MM_UNROLL))
    vmins, vmaxs = lax.fori_loop(0, N_QUERIES // LANES // MM_UNROLL, mm_step,
                                 (vmin0, vmax0))
    vmin = functools.reduce(jnp.minimum, vmins)
    vmax = functools.reduce(jnp.maximum, vmaxs)
    min_x = jnp.broadcast_to(jnp.min(vmin), (LANES,))
    max_x = jnp.broadcast_to(jnp.max(vmax), (LANES,))

    p_head = pts_v[pl.ds(0, LANES)]
    p_tail = pts_v[pl.ds(N_KNOTS - LANES, LANES)]
    v_head = vals_v[pl.ds(0, LANES)]
    v_tail = vals_v[pl.ds(N_KNOTS - LANES, LANES)]

    def splat(vec, lane):
        return jnp.broadcast_to(vec[lane], (LANES,))

    p0 = splat(p_head, 0)
    p1 = splat(p_head, 1)
    pm2 = splat(p_tail, LANES - 2)
    pm1 = splat(p_tail, LANES - 1)
    v0 = splat(v_head, 0)
    v1 = splat(v_head, 1)
    vm2 = splat(v_tail, LANES - 2)
    vm1 = splat(v_tail, LANES - 1)
    lo_val = _vec_interp(min_x, p0, p1, v0, v1)
    hi_val = _vec_interp(max_x, pm2, pm1, vm2, vm1)
    mid0_splat = splat(pts_v[pl.ds(N_KNOTS // 2, LANES)], 0)

    qbase = wid * QPW
    G_UNROLL = 4

    def group(it, carry):
        gbase = it * G_UNROLL
        xqs = [x_v[pl.ds(qbase + (gbase + k) * LANES, LANES)]
               for k in range(G_UNROLL)]
        ss = _lower_bound_multi(pts_v, xqs, mid0_splat)

        pss, yss, pv0s, sm1s = [], [], [], []
        dup = None
        for k in range(G_UNROLL):
            s = ss[k]
            s_c = jnp.minimum(s, N_KNOTS - 1)
            pss.append(plsc.load_gather(pts_v, [s_c]))
            yss.append(plsc.load_gather(vals_v, [s_c]))
            sm1 = jnp.maximum(s - 1, 0)
            sm2 = jnp.maximum(s - 2, 0)
            pv0 = plsc.load_gather(pts_v, [sm1])
            pv1 = plsc.load_gather(pts_v, [sm2])
            sm1s.append(sm1)
            pv0s.append(pv0)
            d = jnp.logical_and(sm1 > 0, pv1 == pv0)
            dup = d if dup is None else jnp.logical_or(dup, d)

        def slow_fn(op):
            pv0_t, sm1_t = op
            fs = _lower_bound_multi(pts_v, list(pv0_t), mid0_splat)
            return tuple(plsc.load_gather(vals_v, [f]) for f in fs)

        def fast_fn(op):
            _, sm1_t = op
            return tuple(plsc.load_gather(vals_v, [i]) for i in sm1_t)

        yfs = lax.cond(jnp.any(dup), slow_fn, fast_fn,
                       (tuple(pv0s), tuple(sm1s)))

        for k in range(G_UNROLL):
            s, xq, ps, ys, pv0, yf = ss[k], xqs[k], pss[k], yss[k], pv0s[k], yfs[k]
            in_range = s < N_KNOTS
            equal = jnp.logical_and(in_range, ps == xq)
            case_b = jnp.logical_and(s == 0, jnp.logical_not(equal))
            case_c = jnp.logical_not(in_range)
            x0 = jnp.where(case_b, min_x, pv0)
            y0 = jnp.where(case_b, lo_val, yf)
            x1 = jnp.where(case_b, p0, jnp.where(case_c, max_x, ps))
            y1 = jnp.where(case_b, v0, jnp.where(case_c, hi_val, ys))
            y = jnp.where(equal, ys,
                          y0 + (y1 - y0) * (xq - x0) * _recip(x1 - x0))
            out_v[pl.ds((gbase + k) * LANES, LANES)] = y
        return carry

    lax.fori_loop(0, GROUPS // G_UNROLL, group, 0)
    pltpu.sync_copy(out_v, out_hbm.at[pl.ds(qbase, QPW)])


_interp_call = functools.partial(
    pl.kernel,
    mesh=plsc.VectorSubcoreMesh(core_axis_name="c", subcore_axis_name="s"),
    out_type=jax.ShapeDtypeStruct((N_QUERIES,), jnp.float32),
    compiler_params=pltpu.CompilerParams(needs_layout_passes=False),
    scratch_types=[
        pltpu.VMEM((N_QUERIES,), jnp.float32),
        pltpu.VMEM((N_KNOTS,), jnp.float32),
        pltpu.VMEM((N_KNOTS,), jnp.float32),
        pltpu.VMEM((QPW,), jnp.float32),
    ],
)(_body)


def kernel(x, points, values):
    pts, vals = lax.sort([points, values], num_keys=1, is_stable=True)
    return _interp_call(x, pts, vals)


# speedup vs baseline: 27.2835x; 1.0457x over previous
import functools

import jax
import jax.numpy as jnp
from jax import lax
from jax.experimental import pallas as pl
from jax.experimental.pallas import tpu as pltpu
from jax.experimental.pallas import tpu_sc as plsc

N_KNOTS = 4096
N_QUERIES = 16384
NC = 2
NS = 16
LANES = 16
NW = NC * NS
QPW = N_QUERIES // NW
GROUPS = QPW // LANES
MM_CHUNK = N_QUERIES // NS
MM_STEPS = MM_CHUNK // LANES
SEARCH_STEPS = 12


def _lower_bound_multi(pts_v, targets, first_probe_splat):
    n = len(targets)
    w = N_KNOTS // 2
    poss = []
    for t in targets:
        less = first_probe_splat < t
        poss.append(jnp.where(less, jnp.int32(w), jnp.int32(0)))
    widths = []
    w //= 2
    while w >= 1:
        widths.append(w)
        w //= 2
    widths.append(1)
    for w in widths:
        for k in range(n):
            probe = poss[k] + (w - 1)
            pv = plsc.load_gather(pts_v, [probe])
            less = pv < targets[k]
            poss[k] = jnp.where(less, poss[k] + w, poss[k])
    return poss


def _recip(d):
    r = 1.0 / d
    r = r * (2.0 - d * r)
    r = r * (2.0 - d * r)
    return r


def _vec_interp(xs, x0, x1, y0, y1):
    delta = x1 - x0
    deg = delta == 0.0
    delta = jnp.where(deg, 1.0, delta)
    return jnp.where(deg, y0, y0 + (y1 - y0) * (xs - x0) * _recip(delta))


def _body(x_hbm, pts_hbm, vals_hbm, out_hbm,
          x_v, pts_v, vals_v, out_v, dma_sem):
    cid = lax.axis_index("c")
    sid = lax.axis_index("s")
    wid = sid * NC + cid

    c1 = pltpu.async_copy(x_hbm, x_v, dma_sem)
    c2 = pltpu.async_copy(pts_hbm, pts_v, dma_sem)
    c3 = pltpu.async_copy(vals_hbm, vals_v, dma_sem)
    c1.wait()
    c2.wait()
    c3.wait()

    MM_UNROLL = 8

    def mm_step(i, carry):
        vmins, vmaxs = carry
        new_mins, new_maxs = [], []
        for k in range(MM_UNROLL):
            xv = x_v[pl.ds((i * MM_UNROLL + k) * LANES, LANES)]
            new_mins.append(jnp.minimum(vmins[k], xv))
            new_maxs.append(jnp.maximum(vmaxs[k], xv))
        return tuple(new_mins), tuple(new_maxs)

    vmin0 = tuple(jnp.full((LANES,), jnp.inf, jnp.float32)
                  for _ in range(MM_UNROLL))
    vmax0 = tuple(jnp.full((LANES,), -jnp.inf, jnp.float32)
                  for _ in range(MM_UNROLL))
    vmins, vmaxs = lax.fori_loop(0, N_QUERIES // LANES // MM_UNROLL, mm_step,
                                 (vmin0, vmax0))
    vmin = functools.reduce(jnp.minimum, vmins)
    vmax = functools.reduce(jnp.maximum, vmaxs)
    min_x = jnp.broadcast_to(jnp.min(vmin), (LANES,))
    max_x = jnp.broadcast_to(jnp.max(vmax), (LANES,))

    p_head = pts_v[pl.ds(0, LANES)]
    p_tail = pts_v[pl.ds(N_KNOTS - LANES, LANES)]
    v_head = vals_v[pl.ds(0, LANES)]
    v_tail = vals_v[pl.ds(N_KNOTS - LANES, LANES)]

    def splat(vec, lane):
        return jnp.broadcast_to(vec[lane], (LANES,))

    p0 = splat(p_head, 0)
    p1 = splat(p_head, 1)
    pm2 = splat(p_tail, LANES - 2)
    pm1 = splat(p_tail, LANES - 1)
    v0 = splat(v_head, 0)
    v1 = splat(v_head, 1)
    vm2 = splat(v_tail, LANES - 2)
    vm1 = splat(v_tail, LANES - 1)
    lo_val = _vec_interp(min_x, p0, p1, v0, v1)
    hi_val = _vec_interp(max_x, pm2, pm1, vm2, vm1)
    first_probe = splat(pts_v[pl.ds(N_KNOTS // 2 - LANES, LANES)], LANES - 1)

    qbase = wid * QPW
    G_UNROLL = 8

    def group(it, carry):
        gbase = it * G_UNROLL
        xqs = [x_v[pl.ds(qbase + (gbase + k) * LANES, LANES)]
               for k in range(G_UNROLL)]
        ss = _lower_bound_multi(pts_v, xqs, first_probe)

        pss, yss, pv0s, sm1s = [], [], [], []
        dup = None
        for k in range(G_UNROLL):
            s = ss[k]
            s_c = jnp.minimum(s, N_KNOTS - 1)
            pss.append(plsc.load_gather(pts_v, [s_c]))
            yss.append(plsc.load_gather(vals_v, [s_c]))
            sm1 = jnp.maximum(s - 1, 0)
            sm2 = jnp.maximum(s - 2, 0)
            pv0 = plsc.load_gather(pts_v, [sm1])
            pv1 = plsc.load_gather(pts_v, [sm2])
            sm1s.append(sm1)
            pv0s.append(pv0)
            d = jnp.logical_and(sm1 > 0, pv1 == pv0)
            dup = d if dup is None else jnp.logical_or(dup, d)

        def slow_fn(op):
            pv0_t, sm1_t = op
            fs = _lower_bound_multi(pts_v, list(pv0_t), first_probe)
            return tuple(plsc.load_gather(vals_v, [f]) for f in fs)

        def fast_fn(op):
            _, sm1_t = op
            return tuple(plsc.load_gather(vals_v, [i]) for i in sm1_t)

        yfs = lax.cond(jnp.any(dup), slow_fn, fast_fn,
                       (tuple(pv0s), tuple(sm1s)))

        for k in range(G_UNROLL):
            s, xq, ps, ys, pv0, yf = ss[k], xqs[k], pss[k], yss[k], pv0s[k], yfs[k]
            in_range = s < N_KNOTS
            equal = jnp.logical_and(in_range, ps == xq)
            case_b = jnp.logical_and(s == 0, jnp.logical_not(equal))
            case_c = jnp.logical_not(in_range)
            x0 = jnp.where(case_b, min_x, pv0)
            y0 = jnp.where(case_b, lo_val, yf)
            x1 = jnp.where(case_b, p0, jnp.where(case_c, max_x, ps))
            y1 = jnp.where(case_b, v0, jnp.where(case_c, hi_val, ys))
            y = jnp.where(equal, ys,
                          y0 + (y1 - y0) * (xq - x0) * _recip(x1 - x0))
            out_v[pl.ds((gbase + k) * LANES, LANES)] = y
        return carry

    lax.fori_loop(0, GROUPS // G_UNROLL, group, 0)
    pltpu.sync_copy(out_v, out_hbm.at[pl.ds(qbase, QPW)])


_interp_call = functools.partial(
    pl.kernel,
    mesh=plsc.VectorSubcoreMesh(core_axis_name="c", subcore_axis_name="s"),
    out_type=jax.ShapeDtypeStruct((N_QUERIES,), jnp.float32),
    compiler_params=pltpu.CompilerParams(needs_layout_passes=False),
    scratch_types=[
        pltpu.VMEM((N_QUERIES,), jnp.float32),
        pltpu.VMEM((N_KNOTS,), jnp.float32),
        pltpu.VMEM((N_KNOTS,), jnp.float32),
        pltpu.VMEM((QPW,), jnp.float32),
        pltpu.SemaphoreType.DMA,
    ],
)(_body)


def kernel(x, points, values):
    pts, vals = lax.sort([points, values], num_keys=1, is_stable=True)
    return _interp_call(x, pts, vals)


# speedup vs baseline: 27.6629x; 1.0139x over previous
import functools

import jax
import jax.numpy as jnp
from jax import lax
from jax.experimental import pallas as pl
from jax.experimental.pallas import tpu as pltpu
from jax.experimental.pallas import tpu_sc as plsc

N_KNOTS = 4096
N_QUERIES = 16384
NC = 1
NS = 16
LANES = 16
NW = NC * NS
QPW = N_QUERIES // NW
GROUPS = QPW // LANES
SEARCH_STEPS = 12


def _lower_bound_multi(pts_v, targets, first_probe_splat):
    n = len(targets)
    w = N_KNOTS // 2
    poss = []
    for t in targets:
        less = first_probe_splat < t
        poss.append(jnp.where(less, jnp.int32(w), jnp.int32(0)))
    widths = []
    w //= 2
    while w >= 1:
        widths.append(w)
        w //= 2
    widths.append(1)
    for w in widths:
        for k in range(n):
            probe = poss[k] + (w - 1)
            pv = plsc.load_gather(pts_v, [probe])
            less = pv < targets[k]
            poss[k] = jnp.where(less, poss[k] + w, poss[k])
    return poss


def _recip(d):
    r = 1.0 / d
    r = r * (2.0 - d * r)
    r = r * (2.0 - d * r)
    return r


def _vec_interp(xs, x0, x1, y0, y1):
    delta = x1 - x0
    deg = delta == 0.0
    delta = jnp.where(deg, 1.0, delta)
    return jnp.where(deg, y0, y0 + (y1 - y0) * (xs - x0) * _recip(delta))


def _body(x_hbm, pts_hbm, vals_hbm, out_hbm,
          x_v, pts_v, vals_v, out_v, dma_sem):
    cid = lax.axis_index("c")
    sid = lax.axis_index("s")
    wid = sid * NC + cid

    c1 = pltpu.async_copy(x_hbm, x_v, dma_sem)
    c2 = pltpu.async_copy(pts_hbm, pts_v, dma_sem)
    c3 = pltpu.async_copy(vals_hbm, vals_v, dma_sem)
    c1.wait()
    c2.wait()
    c3.wait()

    MM_UNROLL = 8

    def mm_step(i, carry):
        vmins, vmaxs = carry
        new_mins, new_maxs = [], []
        for k in range(MM_UNROLL):
            xv = x_v[pl.ds((i * MM_UNROLL + k) * LANES, LANES)]
            new_mins.append(jnp.minimum(vmins[k], xv))
            new_maxs.append(jnp.maximum(vmaxs[k], xv))
        return tuple(new_mins), tuple(new_maxs)

    vmin0 = tuple(jnp.full((LANES,), jnp.inf, jnp.float32)
                  for _ in range(MM_UNROLL))
    vmax0 = tuple(jnp.full((LANES,), -jnp.inf, jnp.float32)
                  for _ in range(MM_UNROLL))
    vmins, vmaxs = lax.fori_loop(0, N_QUERIES // LANES // MM_UNROLL, mm_step,
                                 (vmin0, vmax0))
    vmin = functools.reduce(jnp.minimum, vmins)
    vmax = functools.reduce(jnp.maximum, vmaxs)
    min_x = jnp.broadcast_to(jnp.min(vmin), (LANES,))
    max_x = jnp.broadcast_to(jnp.max(vmax), (LANES,))

    p_head = pts_v[pl.ds(0, LANES)]
    p_tail = pts_v[pl.ds(N_KNOTS - LANES, LANES)]
    v_head = vals_v[pl.ds(0, LANES)]
    v_tail = vals_v[pl.ds(N_KNOTS - LANES, LANES)]

    def splat(vec, lane):
        return jnp.broadcast_to(vec[lane], (LANES,))

    p0 = splat(p_head, 0)
    p1 = splat(p_head, 1)
    pm2 = splat(p_tail, LANES - 2)
    pm1 = splat(p_tail, LANES - 1)
    v0 = splat(v_head, 0)
    v1 = splat(v_head, 1)
    vm2 = splat(v_tail, LANES - 2)
    vm1 = splat(v_tail, LANES - 1)
    lo_val = _vec_interp(min_x, p0, p1, v0, v1)
    hi_val = _vec_interp(max_x, pm2, pm1, vm2, vm1)
    first_probe = splat(pts_v[pl.ds(N_KNOTS // 2 - LANES, LANES)], LANES - 1)

    qbase = wid * QPW
    G_UNROLL = 16

    def group(it, carry):
        gbase = it * G_UNROLL
        xqs = [x_v[pl.ds(qbase + (gbase + k) * LANES, LANES)]
               for k in range(G_UNROLL)]
        ss = _lower_bound_multi(pts_v, xqs, first_probe)

        pss, yss, pv0s, sm1s = [], [], [], []
        dup = None
        for k in range(G_UNROLL):
            s = ss[k]
            s_c = jnp.minimum(s, N_KNOTS - 1)
            pss.append(plsc.load_gather(pts_v, [s_c]))
            yss.append(plsc.load_gather(vals_v, [s_c]))
            sm1 = jnp.maximum(s - 1, 0)
            sm2 = jnp.maximum(s - 2, 0)
            pv0 = plsc.load_gather(pts_v, [sm1])
            pv1 = plsc.load_gather(pts_v, [sm2])
            sm1s.append(sm1)
            pv0s.append(pv0)
            d = jnp.logical_and(sm1 > 0, pv1 == pv0)
            dup = d if dup is None else jnp.logical_or(dup, d)

        def slow_fn(op):
            pv0_t, sm1_t = op
            fs = _lower_bound_multi(pts_v, list(pv0_t), first_probe)
            return tuple(plsc.load_gather(vals_v, [f]) for f in fs)

        def fast_fn(op):
            _, sm1_t = op
            return tuple(plsc.load_gather(vals_v, [i]) for i in sm1_t)

        yfs = lax.cond(jnp.any(dup), slow_fn, fast_fn,
                       (tuple(pv0s), tuple(sm1s)))

        for k in range(G_UNROLL):
            s, xq, ps, ys, pv0, yf = ss[k], xqs[k], pss[k], yss[k], pv0s[k], yfs[k]
            in_range = s < N_KNOTS
            equal = jnp.logical_and(in_range, ps == xq)
            case_b = jnp.logical_and(s == 0, jnp.logical_not(equal))
            case_c = jnp.logical_not(in_range)
            x0 = jnp.where(case_b, min_x, pv0)
            y0 = jnp.where(case_b, lo_val, yf)
            x1 = jnp.where(case_b, p0, jnp.where(case_c, max_x, ps))
            y1 = jnp.where(case_b, v0, jnp.where(case_c, hi_val, ys))
            y = jnp.where(equal, ys,
                          y0 + (y1 - y0) * (xq - x0) * _recip(x1 - x0))
            out_v[pl.ds((gbase + k) * LANES, LANES)] = y
        return carry

    lax.fori_loop(0, GROUPS // G_UNROLL, group, 0)
    pltpu.sync_copy(out_v, out_hbm.at[pl.ds(qbase, QPW)])


_interp_call = functools.partial(
    pl.kernel,
    mesh=plsc.VectorSubcoreMesh(core_axis_name="c", subcore_axis_name="s",
                                num_cores=NC),
    out_type=jax.ShapeDtypeStruct((N_QUERIES,), jnp.float32),
    compiler_params=pltpu.CompilerParams(needs_layout_passes=False),
    scratch_types=[
        pltpu.VMEM((N_QUERIES,), jnp.float32),
        pltpu.VMEM((N_KNOTS,), jnp.float32),
        pltpu.VMEM((N_KNOTS,), jnp.float32),
        pltpu.VMEM((QPW,), jnp.float32),
        pltpu.SemaphoreType.DMA,
    ],
)(_body)


def kernel(x, points, values):
    pts, vals = lax.sort([points, values], num_keys=1, is_stable=True)
    return _interp_call(x, pts, vals)


# speedup vs baseline: 28.7470x; 1.0392x over previous
import functools

import jax
import jax.numpy as jnp
from jax import lax
from jax.experimental import pallas as pl
from jax.experimental.pallas import tpu as pltpu
from jax.experimental.pallas import tpu_sc as plsc

N_KNOTS = 4096
N_QUERIES = 16384
NC = 1
NS = 16
LANES = 16
NW = NC * NS
QPW = N_QUERIES // NW
GROUPS = QPW // LANES
SEARCH_STEPS = 12


COARSE = 16
BUCKET = N_KNOTS // COARSE


def _lower_bound_multi(pts_v, targets, thresholds):
    n = len(targets)
    poss = []
    for t in targets:
        c = None
        for thr in thresholds:
            b = jnp.where(thr < t, jnp.int32(BUCKET), jnp.int32(0))
            c = b if c is None else c + b
        poss.append(c)
    widths = []
    w = BUCKET // 2
    while w >= 1:
        widths.append(w)
        w //= 2
    widths.append(1)
    for w in widths:
        for k in range(n):
            probe = poss[k] + (w - 1)
            pv = plsc.load_gather(pts_v, [probe])
            less = pv < targets[k]
            poss[k] = jnp.where(less, poss[k] + w, poss[k])
    return poss


def _recip(d):
    r = 1.0 / d
    r = r * (2.0 - d * r)
    r = r * (2.0 - d * r)
    return r


def _vec_interp(xs, x0, x1, y0, y1):
    delta = x1 - x0
    deg = delta == 0.0
    delta = jnp.where(deg, 1.0, delta)
    return jnp.where(deg, y0, y0 + (y1 - y0) * (xs - x0) * _recip(delta))


def _body(x_hbm, pts_hbm, vals_hbm, out_hbm,
          x_v, pts_v, vals_v, out_v, dma_sem):
    cid = lax.axis_index("c")
    sid = lax.axis_index("s")
    wid = sid * NC + cid

    c1 = pltpu.async_copy(x_hbm, x_v, dma_sem)
    c2 = pltpu.async_copy(pts_hbm, pts_v, dma_sem)
    c3 = pltpu.async_copy(vals_hbm, vals_v, dma_sem)
    c1.wait()
    c2.wait()
    c3.wait()

    MM_UNROLL = 8

    def mm_step(i, carry):
        vmins, vmaxs = carry
        new_mins, new_maxs = [], []
        for k in range(MM_UNROLL):
            xv = x_v[pl.ds((i * MM_UNROLL + k) * LANES, LANES)]
            new_mins.append(jnp.minimum(vmins[k], xv))
            new_maxs.append(jnp.maximum(vmaxs[k], xv))
        return tuple(new_mins), tuple(new_maxs)

    vmin0 = tuple(jnp.full((LANES,), jnp.inf, jnp.float32)
                  for _ in range(MM_UNROLL))
    vmax0 = tuple(jnp.full((LANES,), -jnp.inf, jnp.float32)
                  for _ in range(MM_UNROLL))
    vmins, vmaxs = lax.fori_loop(0, N_QUERIES // LANES // MM_UNROLL, mm_step,
                                 (vmin0, vmax0))
    vmin = functools.reduce(jnp.minimum, vmins)
    vmax = functools.reduce(jnp.maximum, vmaxs)
    min_x = jnp.broadcast_to(jnp.min(vmin), (LANES,))
    max_x = jnp.broadcast_to(jnp.max(vmax), (LANES,))

    p_head = pts_v[pl.ds(0, LANES)]
    p_tail = pts_v[pl.ds(N_KNOTS - LANES, LANES)]
    v_head = vals_v[pl.ds(0, LANES)]
    v_tail = vals_v[pl.ds(N_KNOTS - LANES, LANES)]

    def splat(vec, lane):
        return jnp.broadcast_to(vec[lane], (LANES,))

    p0 = splat(p_head, 0)
    p1 = splat(p_head, 1)
    pm2 = splat(p_tail, LANES - 2)
    pm1 = splat(p_tail, LANES - 1)
    v0 = splat(v_head, 0)
    v1 = splat(v_head, 1)
    vm2 = splat(v_tail, LANES - 2)
    vm1 = splat(v_tail, LANES - 1)
    lo_val = _vec_interp(min_x, p0, p1, v0, v1)
    hi_val = _vec_interp(max_x, pm2, pm1, vm2, vm1)
    thresholds = [
        splat(pts_v[pl.ds(k * BUCKET - LANES, LANES)], LANES - 1)
        for k in range(1, COARSE)
    ]

    qbase = wid * QPW
    G_UNROLL = 8

    def group(it, carry):
        gbase = it * G_UNROLL
        xqs = [x_v[pl.ds(qbase + (gbase + k) * LANES, LANES)]
               for k in range(G_UNROLL)]
        ss = _lower_bound_multi(pts_v, xqs, thresholds)

        pss, yss, pv0s, sm1s = [], [], [], []
        dup = None
        for k in range(G_UNROLL):
            s = ss[k]
            s_c = jnp.minimum(s, N_KNOTS - 1)
            pss.append(plsc.load_gather(pts_v, [s_c]))
            yss.append(plsc.load_gather(vals_v, [s_c]))
            sm1 = jnp.maximum(s - 1, 0)
            sm2 = jnp.maximum(s - 2, 0)
            pv0 = plsc.load_gather(pts_v, [sm1])
            pv1 = plsc.load_gather(pts_v, [sm2])
            sm1s.append(sm1)
            pv0s.append(pv0)
            d = jnp.logical_and(sm1 > 0, pv1 == pv0)
            dup = d if dup is None else jnp.logical_or(dup, d)

        def slow_fn(op):
            pv0_t, sm1_t = op
            fs = _lower_bound_multi(pts_v, list(pv0_t), thresholds)
            return tuple(plsc.load_gather(vals_v, [f]) for f in fs)

        def fast_fn(op):
            _, sm1_t = op
            return tuple(plsc.load_gather(vals_v, [i]) for i in sm1_t)

        yfs = lax.cond(jnp.any(dup), slow_fn, fast_fn,
                       (tuple(pv0s), tuple(sm1s)))

        for k in range(G_UNROLL):
            s, xq, ps, ys, pv0, yf = ss[k], xqs[k], pss[k], yss[k], pv0s[k], yfs[k]
            in_range = s < N_KNOTS
            equal = jnp.logical_and(in_range, ps == xq)
            case_b = jnp.logical_and(s == 0, jnp.logical_not(equal))
            case_c = jnp.logical_not(in_range)
            x0 = jnp.where(case_b, min_x, pv0)
            y0 = jnp.where(case_b, lo_val, yf)
            x1 = jnp.where(case_b, p0, jnp.where(case_c, max_x, ps))
            y1 = jnp.where(case_b, v0, jnp.where(case_c, hi_val, ys))
            y = jnp.where(equal, ys,
                          y0 + (y1 - y0) * (xq - x0) * _recip(x1 - x0))
            out_v[pl.ds((gbase + k) * LANES, LANES)] = y
        return carry

    lax.fori_loop(0, GROUPS // G_UNROLL, group, 0)
    pltpu.sync_copy(out_v, out_hbm.at[pl.ds(qbase, QPW)])


_interp_call = functools.partial(
    pl.kernel,
    mesh=plsc.VectorSubcoreMesh(core_axis_name="c", subcore_axis_name="s",
                                num_cores=NC),
    out_type=jax.ShapeDtypeStruct((N_QUERIES,), jnp.float32),
    compiler_params=pltpu.CompilerParams(needs_layout_passes=False),
    scratch_types=[
        pltpu.VMEM((N_QUERIES,), jnp.float32),
        pltpu.VMEM((N_KNOTS,), jnp.float32),
        pltpu.VMEM((N_KNOTS,), jnp.float32),
        pltpu.VMEM((QPW,), jnp.float32),
        pltpu.SemaphoreType.DMA,
    ],
)(_body)


def kernel(x, points, values):
    pts, vals = lax.sort([points, values], num_keys=1, is_stable=True)
    return _interp_call(x, pts, vals)


# speedup vs baseline: 28.8104x; 1.0022x over previous
import functools

import jax
import jax.numpy as jnp
from jax import lax
from jax.experimental import pallas as pl
from jax.experimental.pallas import tpu as pltpu
from jax.experimental.pallas import tpu_sc as plsc

N_KNOTS = 4096
N_QUERIES = 16384
NC = 1
NS = 16
LANES = 16
NW = NC * NS
QPW = N_QUERIES // NW
GROUPS = QPW // LANES

COARSE = 16
BUCKET = N_KNOTS // COARSE


def _lower_bound_multi(pts_v, targets, thresholds):
    n = len(targets)
    poss = []
    for t in targets:
        c = None
        for thr in thresholds:
            b = jnp.where(thr < t, jnp.int32(BUCKET), jnp.int32(0))
            c = b if c is None else c + b
        poss.append(c)
    widths = []
    w = BUCKET // 2
    while w >= 1:
        widths.append(w)
        w //= 2
    widths.append(1)
    for w in widths:
        for k in range(n):
            probe = poss[k] + (w - 1)
            pv = plsc.load_gather(pts_v, [probe])
            less = pv < targets[k]
            poss[k] = jnp.where(less, poss[k] + w, poss[k])
    return poss


def _recip(d):
    r = 1.0 / d
    r = r * (2.0 - d * r)
    r = r * (2.0 - d * r)
    return r


def _vec_interp(xs, x0, x1, y0, y1):
    delta = x1 - x0
    deg = delta == 0.0
    delta = jnp.where(deg, 1.0, delta)
    return jnp.where(deg, y0, y0 + (y1 - y0) * (xs - x0) * _recip(delta))


def _body(x_hbm, pts_hbm, vals_hbm, out_hbm,
          x_v, pts_v, vals_v, out_v, dma_sem):
    cid = lax.axis_index("c")
    sid = lax.axis_index("s")
    wid = sid * NC + cid

    c1 = pltpu.async_copy(x_hbm, x_v, dma_sem)
    c2 = pltpu.async_copy(pts_hbm, pts_v, dma_sem)
    c3 = pltpu.async_copy(vals_hbm, vals_v, dma_sem)
    c1.wait()
    c2.wait()
    c3.wait()

    MM_UNROLL = 8

    def mm_step(i, carry):
        vmins, vmaxs = carry
        new_mins, new_maxs = [], []
        for k in range(MM_UNROLL):
            xv = x_v[pl.ds((i * MM_UNROLL + k) * LANES, LANES)]
            new_mins.append(jnp.minimum(vmins[k], xv))
            new_maxs.append(jnp.maximum(vmaxs[k], xv))
        return tuple(new_mins), tuple(new_maxs)

    vmin0 = tuple(jnp.full((LANES,), jnp.inf, jnp.float32)
                  for _ in range(MM_UNROLL))
    vmax0 = tuple(jnp.full((LANES,), -jnp.inf, jnp.float32)
                  for _ in range(MM_UNROLL))
    vmins, vmaxs = lax.fori_loop(0, N_QUERIES // LANES // MM_UNROLL, mm_step,
                                 (vmin0, vmax0))
    vmin = functools.reduce(jnp.minimum, vmins)
    vmax = functools.reduce(jnp.maximum, vmaxs)
    min_x = jnp.broadcast_to(jnp.min(vmin), (LANES,))
    max_x = jnp.broadcast_to(jnp.max(vmax), (LANES,))

    p_head = pts_v[pl.ds(0, LANES)]
    p_tail = pts_v[pl.ds(N_KNOTS - LANES, LANES)]
    v_head = vals_v[pl.ds(0, LANES)]
    v_tail = vals_v[pl.ds(N_KNOTS - LANES, LANES)]

    def splat(vec, lane):
        return jnp.broadcast_to(vec[lane], (LANES,))

    p0 = splat(p_head, 0)
    p1 = splat(p_head, 1)
    pm2 = splat(p_tail, LANES - 2)
    pm1 = splat(p_tail, LANES - 1)
    v0 = splat(v_head, 0)
    v1 = splat(v_head, 1)
    vm2 = splat(v_tail, LANES - 2)
    vm1 = splat(v_tail, LANES - 1)
    lo_val = _vec_interp(min_x, p0, p1, v0, v1)
    hi_val = _vec_interp(max_x, pm2, pm1, vm2, vm1)
    thresholds = [
        splat(pts_v[pl.ds(k * BUCKET - LANES, LANES)], LANES - 1)
        for k in range(1, COARSE)
    ]

    qbase = wid * QPW
    G_UNROLL = 8

    def group(it, carry):
        gbase = it * G_UNROLL
        xqs = [x_v[pl.ds(qbase + (gbase + k) * LANES, LANES)]
               for k in range(G_UNROLL)]
        ss = _lower_bound_multi(pts_v, xqs, thresholds)

        pss, yss, pv0s, sm1s = [], [], [], []
        dup = None
        for k in range(G_UNROLL):
            s = ss[k]
            s_c = jnp.minimum(s, N_KNOTS - 1)
            pss.append(plsc.load_gather(pts_v, [s_c]))
            yss.append(plsc.load_gather(vals_v, [s_c]))
            sm1 = jnp.maximum(s - 1, 0)
            sm2 = jnp.maximum(s - 2, 0)
            pv0 = plsc.load_gather(pts_v, [sm1])
            pv1 = plsc.load_gather(pts_v, [sm2])
            sm1s.append(sm1)
            pv0s.append(pv0)
            d = jnp.logical_and(sm1 > 0, pv1 == pv0)
            dup = d if dup is None else jnp.logical_or(dup, d)

        def slow_fn(op):
            pv0_t, sm1_t = op
            fs = _lower_bound_multi(pts_v, list(pv0_t), thresholds)
            return tuple(plsc.load_gather(vals_v, [f]) for f in fs)

        def fast_fn(op):
            _, sm1_t = op
            return tuple(plsc.load_gather(vals_v, [i]) for i in sm1_t)

        yfs = lax.cond(jnp.any(dup), slow_fn, fast_fn,
                       (tuple(pv0s), tuple(sm1s)))

        for k in range(G_UNROLL):
            s, xq, ps, ys, pv0, yf = ss[k], xqs[k], pss[k], yss[k], pv0s[k], yfs[k]
            in_range = s < N_KNOTS
            equal = jnp.logical_and(in_range, ps == xq)
            case_b = jnp.logical_and(s == 0, jnp.logical_not(equal))
            case_c = jnp.logical_not(in_range)
            x0 = jnp.where(case_b, min_x, pv0)
            y0 = jnp.where(case_b, lo_val, yf)
            x1 = jnp.where(case_b, p0, jnp.where(case_c, max_x, ps))
            y1 = jnp.where(case_b, v0, jnp.where(case_c, hi_val, ys))
            y = jnp.where(equal, ys,
                          y0 + (y1 - y0) * (xq - x0) * _recip(x1 - x0))
            out_v[pl.ds((gbase + k) * LANES, LANES)] = y
        return carry

    lax.fori_loop(0, GROUPS // G_UNROLL, group, 0)
    pltpu.sync_copy(out_v, out_hbm.at[pl.ds(qbase, QPW)])


_interp_call = functools.partial(
    pl.kernel,
    mesh=plsc.VectorSubcoreMesh(core_axis_name="c", subcore_axis_name="s",
                                num_cores=NC),
    out_type=jax.ShapeDtypeStruct((N_QUERIES,), jnp.float32),
    compiler_params=pltpu.CompilerParams(needs_layout_passes=False),
    scratch_types=[
        pltpu.VMEM((N_QUERIES,), jnp.float32),
        pltpu.VMEM((N_KNOTS,), jnp.float32),
        pltpu.VMEM((N_KNOTS,), jnp.float32),
        pltpu.VMEM((QPW,), jnp.float32),
        pltpu.SemaphoreType.DMA,
    ],
)(_body)


def kernel(x, points, values):
    pts, vals = lax.sort([points, values], num_keys=1, is_stable=True)
    return _interp_call(x, pts, vals)
